# baseline (speedup 1.0000x reference)
import functools

import jax
import jax.numpy as jnp
import numpy as np
from jax import lax
from jax.experimental import pallas as pl
from jax.experimental.pallas import tpu as pltpu

NCLS = 19
RPS = 8
CPAD = 24


def _x_weight_matrix(in_w, out_w):
    xs = np.linspace(0.0, in_w - 1.0, out_w)
    x0 = np.floor(xs).astype(np.int64)
    x1 = np.minimum(x0 + 1, in_w - 1)
    wx = xs - np.floor(xs)
    wm = np.zeros((in_w, out_w), np.float64)
    np.add.at(wm, (x0, np.arange(out_w)), 1.0 - wx)
    np.add.at(wm, (x1, np.arange(out_w)), wx)
    return wm.astype(np.float32)


def _fused_body(wx_ref, s_sm, t_sm, s_feat, t_feat, out_ref,
                s_scr, t_scr, acc_sum_s, acc_misc_s, acc_sum_t, acc_misc_t,
                sem, *, in_h, out_h, out_w, nb, fdim):
    b = pl.program_id(0)
    t = pl.program_id(1)
    nsteps = pl.num_programs(0) * nb
    step = b * nb + t
    first = step == 0
    last = step == nsteps - 1
    num = in_h - 1
    den = out_h - 1
    pb = RPS * out_w

    def row_dmas(bb, tt, buf, sm_ref, scr_ref, dom):
        copies = []
        for r in range(RPS):
            i = tt * RPS + r
            y0 = (i * num) // den
            y1 = jnp.minimum(y0 + 1, in_h - 1)
            copies.append(pltpu.make_async_copy(
                sm_ref.at[bb, :, y0, :], scr_ref.at[buf, 2 * r],
                sem.at[buf, dom]))
            copies.append(pltpu.make_async_copy(
                sm_ref.at[bb, :, y1, :], scr_ref.at[buf, 2 * r + 1],
                sem.at[buf, dom]))
        return copies

    def issue(bb, tt, buf):
        for c in row_dmas(bb, tt, buf, s_sm, s_scr, 0):
            c.start()
        for c in row_dmas(bb, tt, buf, t_sm, t_scr, 1):
            c.start()

    def drain(bb, tt, buf):
        for c in row_dmas(bb, tt, buf, s_sm, s_scr, 0):
            c.wait()
        for c in row_dmas(bb, tt, buf, t_sm, t_scr, 1):
            c.wait()

    cur = lax.rem(step, 4)

    @pl.when(first)
    def _prologue():
        issue(b, t, cur)
        issue(0, 1, 1)
        issue(0, 2, 2)

    @pl.when(step + 3 < nsteps)
    def _prefetch():
        nstep = step + 3
        bn = nstep // nb
        tn = lax.rem(nstep, nb)
        issue(bn, tn, lax.rem(nstep, 4))

    drain(b, t, cur)

    @pl.when(first)
    def _init():
        acc_sum_s[...] = jnp.zeros_like(acc_sum_s)
        acc_misc_s[...] = jnp.zeros_like(acc_misc_s)
        acc_sum_t[...] = jnp.zeros_like(acc_sum_t)
        acc_misc_t[...] = jnp.zeros_like(acc_misc_t)

    zpad = jnp.zeros((CPAD - NCLS, s_scr.shape[-1]), jnp.float32)
    wmat = wx_ref[...]
    for scr_ref, feat_ref, acc_sum, acc_misc in (
            (s_scr, s_feat, acc_sum_s, acc_misc_s),
            (t_scr, t_feat, acc_sum_t, acc_misc_t)):
        pieces = []
        for r in range(RPS):
            i = t * RPS + r
            y0 = (i * num) // den
            rem = i * num - y0 * den
            wy = rem.astype(jnp.float32) / float(den)
            top = scr_ref[cur, 2 * r]
            bot = scr_ref[cur, 2 * r + 1]
            pieces.append(top * (1.0 - wy) + bot * wy)
            pieces.append(zpad)
        stacked = jnp.concatenate(pieces, axis=0)
        vals = jax.lax.dot_general(
            stacked, wmat,
            dimension_numbers=(((1,), (0,)), ((), ())),
            preferred_element_type=jnp.float32,
            precision=jax.lax.Precision.DEFAULT,
        )
        oh_pieces = []
        for r in range(RPS):
            blk = lax.slice(vals, (CPAD * r, 0), (CPAD * r + CPAD, out_w))
            sub = lax.broadcasted_iota(jnp.int32, (CPAD, out_w), 0)
            real = sub < NCLS
            m = jnp.max(jnp.where(real, blk, -1.0), axis=0, keepdims=True)
            idx = jnp.min(jnp.where((blk >= m) & real, sub, NCLS), axis=0,
                          keepdims=True)
            ohr = (lax.broadcasted_iota(jnp.int32, (NCLS, out_w), 0)
                   == idx).astype(jnp.float32)
            oh_pieces.append(ohr)
        ohT = jnp.concatenate(oh_pieces, axis=1)

        feat4 = feat_ref[0]
        feat2 = feat4.reshape(feat4.shape[0], pb)
        acc_sum[...] += jax.lax.dot_general(
            feat2, ohT,
            dimension_numbers=(((1,), (1,)), ((), ())),
            preferred_element_type=jnp.float32,
            precision=jax.lax.Precision.DEFAULT,
        )
        csq8 = jnp.sum(feat4 * feat4, axis=0)
        colsq = csq8.reshape(1, pb)
        ios = lax.broadcasted_iota(jnp.int32, (8, pb), 0)
        extra = jnp.where(ios == 0, colsq,
                          jnp.where(ios == 1, 1.0, 0.0))
        acc_misc[...] += jax.lax.dot_general(
            extra, ohT,
            dimension_numbers=(((1,), (1,)), ((), ())),
            preferred_element_type=jnp.float32,
            precision=jax.lax.Precision.DEFAULT,
        )

    @pl.when(last)
    def _finish():
        fdim_f = float(fdim)
        io8 = lax.broadcasted_iota(jnp.int32, (8, NCLS), 0)

        def row(m_ref, r):
            return jnp.sum(jnp.where(io8 == r, m_ref[...], 0.0), axis=0,
                           keepdims=True)

        sum_s = acc_sum_s[...]
        sum_t = acc_sum_t[...]
        sumsq_s = row(acc_misc_s, 0)
        cnt_s = row(acc_misc_s, 1)
        sumsq_t = row(acc_misc_t, 0)
        cnt_t = row(acc_misc_t, 1)

        cnt_tot = cnt_s + cnt_t
        valid = cnt_tot > 0.0
        cent = (sum_s + sum_t) / jnp.maximum(cnt_tot, 1.0)
        cn2 = jnp.sum(cent * cent, axis=0, keepdims=True)

        def f2c(sum_d, sumsq_d, cnt_d):
            dot_cs = jnp.sum(cent * sum_d, axis=0, keepdims=True)
            ssq = jnp.maximum(sumsq_d - 2.0 * dot_cs + cnt_d * cn2, 0.0)
            ok = cnt_d > 0.0
            nrm = jnp.sqrt(jnp.where(ok, ssq, 1.0))
            dist = nrm / jnp.maximum(cnt_d * fdim_f, 1.0)
            nseen = jnp.sum(jnp.where(ok, 1.0, 0.0))
            return jnp.sum(jnp.where(ok, dist, 0.0)) / jnp.maximum(nseen, 1.0)

        loss_s = f2c(sum_s, sumsq_s, cnt_s)
        loss_t = f2c(sum_t, sumsq_t, cnt_t)

        centv = jnp.where(valid, cent, 0.0)
        n2 = jnp.sum(centv * centv, axis=0, keepdims=True)
        iota_l = lax.broadcasted_iota(jnp.int32, (1, NCLS), 1)
        iota_fl = lax.broadcasted_iota(jnp.int32, (fdim, NCLS), 1)
        ssq_vec = jnp.zeros((1, NCLS), jnp.float32)
        for i in range(NCLS):
            ci = jnp.sum(jnp.where(iota_fl == i, centv, 0.0), axis=1,
                         keepdims=True)
            gi = jnp.sum(ci * centv, axis=0, keepdims=True)
            n2_i = jnp.sum(jnp.where(iota_l == i, n2, 0.0))
            sqrow = n2 + n2_i - 2.0 * gi
            contrib = jnp.sum(jnp.where((iota_l != i) & valid, sqrow, 0.0))
            ssq_vec = ssq_vec + jnp.where(iota_l == i, contrib, 0.0)

        nvalid = jnp.sum(jnp.where(valid, 1.0, 0.0))
        denom = jnp.maximum((nvalid - 1.0) * fdim_f, 1.0)
        nrm_i = jnp.sqrt(jnp.where(valid, ssq_vec, 1.0))
        dist_i = nrm_i / denom
        c2c = jnp.sum(jnp.where(valid, dist_i, 0.0)) / jnp.maximum(nvalid, 1.0)

        out_ref[...] = jnp.broadcast_to(loss_s + loss_t + c2c, (1, 1))


def kernel(source_feat, source_softmax, target_feat, target_softmax):
    B, F, h, w = source_feat.shape
    _, C, in_h, in_w = source_softmax.shape
    nb = h // RPS

    wmat = jnp.asarray(_x_weight_matrix(in_w, w))

    feat_spec = pl.BlockSpec((1, F, RPS, w), lambda b, t: (b, 0, t, 0))
    any_spec = pl.BlockSpec(memory_space=pl.ANY)

    loss = pl.pallas_call(
        functools.partial(_fused_body, in_h=in_h, out_h=h, out_w=w, nb=nb,
                          fdim=F),
        grid=(B, nb),
        in_specs=[pl.BlockSpec((in_w, w), lambda b, t: (0, 0)),
                  any_spec, any_spec, feat_spec, feat_spec],
        out_specs=pl.BlockSpec((1, 1), lambda b, t: (0, 0)),
        out_shape=jax.ShapeDtypeStruct((1, 1), jnp.float32),
        scratch_shapes=[
            pltpu.VMEM((4, 2 * RPS, C, in_w), jnp.float32),
            pltpu.VMEM((4, 2 * RPS, C, in_w), jnp.float32),
            pltpu.VMEM((F, NCLS), jnp.float32),
            pltpu.VMEM((8, NCLS), jnp.float32),
            pltpu.VMEM((F, NCLS), jnp.float32),
            pltpu.VMEM((8, NCLS), jnp.float32),
            pltpu.SemaphoreType.DMA((4, 2)),
        ],
    )(wmat, source_softmax, target_softmax, source_feat, target_feat)
    return loss[0, 0]
